# Initial kernel scaffold; baseline (speedup 1.0000x reference)
import jax
import jax.numpy as jnp
from jax.experimental import pallas as pl


def kernel(x, edge_index, W_self1, W_neigh1, b1, W_self2, W_neigh2, b2, W_self3, W_neigh3, b3):
    raise NotImplementedError("write your pallas kernel here")


# speedup vs baseline: 3.6151x; 3.6151x over previous
import jax
import jax.numpy as jnp
from jax import lax
from jax.experimental import pallas as pl
from jax.experimental.pallas import tpu as pltpu
from jax.experimental.pallas import tpu_sc as plsc

N = 10000
D = 128
HD = D // 2
NUM_EDGES = 320000

NC = 2
NS = 16

CHUNK = 128
CHUNKS = 160
HALF = CHUNKS // 2
EDGES_PER_W = CHUNK * CHUNKS
E_PAD = NS * EDGES_PER_W
N_PAD = 10240
STRIPE = N_PAD // NS
DEG_W = 16

_MESH = plsc.VectorSubcoreMesh(core_axis_name="c", subcore_axis_name="s")


def _zero_vmem(ref, nwords):
    cols = ref.shape[-1]
    per_row = cols // 16

    def body(i, _):
        r = lax.div(i, per_row)
        c = lax.rem(i, per_row) * 16
        ref[r, pl.ds(c, 16)] = jnp.zeros((16,), jnp.float32)
        return 0

    lax.fori_loop(0, nwords // 16, body, 0)


def _fill_ones(ref):
    rows, cols = ref.shape
    per_row = cols // 16

    def body(i, _):
        r = lax.div(i, per_row)
        c = lax.rem(i, per_row) * 16
        ref[r, pl.ds(c, 16)] = jnp.ones((16,), jnp.float32)
        return 0

    lax.fori_loop(0, rows * per_row, body, 0)


def _make_agg():
    out_type = jax.ShapeDtypeStruct((N_PAD, NC, HD), jnp.float32)
    scratch = [
        pltpu.VMEM((CHUNKS, CHUNK), jnp.int32),
        pltpu.VMEM((CHUNKS, CHUNK), jnp.int32),
        pltpu.VMEM((2, CHUNK, HD), jnp.float32),
        pltpu.VMEM_SHARED((N_PAD, HD), jnp.float32),
        pltpu.SemaphoreType.DMA,
        pltpu.SemaphoreType.DMA,
    ]

    def body(t_hbm, srcs_hbm, dsts_hbm, out_hbm, idx_s, idx_d, rows, acc,
             sem0, sem1):
        cid = lax.axis_index("c")
        sid = lax.axis_index("s")

        pltpu.sync_copy(srcs_hbm.at[sid], idx_s)
        pltpu.sync_copy(dsts_hbm.at[sid], idx_d)

        cvec = jnp.full((16,), 1, jnp.int32) * cid

        def xform(i, _):
            r = lax.div(i, CHUNK // 16)
            c = lax.rem(i, CHUNK // 16) * 16
            v = idx_s[r, pl.ds(c, 16)]
            idx_s[r, pl.ds(c, 16)] = v * 2 + cvec
            return 0

        lax.fori_loop(0, CHUNKS * CHUNK // 16, xform, 0)

        _zero_vmem(rows.at[0], CHUNK * HD)
        for k in range(STRIPE // CHUNK):
            pltpu.sync_copy(rows.at[0], acc.at[pl.ds(sid * STRIPE + k * CHUNK, CHUNK)])
        plsc.subcore_barrier()

        pltpu.async_copy(t_hbm.at[idx_s.at[0]], rows.at[0], sem0)

        def step(i, _):
            j0 = 2 * i
            j1 = j0 + 1
            pltpu.async_copy(t_hbm.at[idx_s.at[j1]], rows.at[1], sem1)
            pltpu.make_async_copy(t_hbm.at[idx_s.at[j0]], rows.at[0], sem0).wait()
            pltpu.sync_copy(rows.at[0], acc.at[idx_d.at[j0]], add=True)

            @pl.when(i + 1 < HALF)
            def _():
                pltpu.async_copy(t_hbm.at[idx_s.at[j0 + 2]], rows.at[0], sem0)

            pltpu.make_async_copy(t_hbm.at[idx_s.at[j1]], rows.at[1], sem1).wait()
            pltpu.sync_copy(rows.at[1], acc.at[idx_d.at[j1]], add=True)
            return 0

        lax.fori_loop(0, HALF, step, 0)
        plsc.subcore_barrier()

        pltpu.sync_copy(acc.at[pl.ds(sid * STRIPE, STRIPE)],
                        out_hbm.at[pl.ds(sid * STRIPE, STRIPE), cid])

    return pl.kernel(body, out_type=out_type, mesh=_MESH,
                     scratch_types=tuple(scratch),
                     compiler_params=pltpu.CompilerParams(
                         use_tc_tiling_on_sc=False))


def _make_deg():
    out_type = jax.ShapeDtypeStruct((NC, N_PAD, DEG_W), jnp.float32)
    scratch = [
        pltpu.VMEM((HALF, CHUNK), jnp.int32),
        pltpu.VMEM((CHUNK, DEG_W), jnp.float32),
        pltpu.VMEM((STRIPE, DEG_W), jnp.float32),
        pltpu.VMEM_SHARED((N_PAD, DEG_W), jnp.float32),
    ]

    def body(dsts_hbm, deg_hbm, idx_d, ones_v, degz, dacc):
        cid = lax.axis_index("c")
        sid = lax.axis_index("s")

        pltpu.sync_copy(dsts_hbm.at[sid, pl.ds(cid * HALF, HALF)], idx_d)
        _fill_ones(ones_v)
        _zero_vmem(degz, STRIPE * DEG_W)
        pltpu.sync_copy(degz, dacc.at[pl.ds(sid * STRIPE, STRIPE)])
        plsc.subcore_barrier()

        def step(j, _):
            pltpu.sync_copy(ones_v, dacc.at[idx_d.at[j]], add=True)
            return 0

        lax.fori_loop(0, HALF, step, 0)
        plsc.subcore_barrier()
        pltpu.sync_copy(dacc.at[pl.ds(sid * STRIPE, STRIPE)],
                        deg_hbm.at[cid, pl.ds(sid * STRIPE, STRIPE)])

    return pl.kernel(body, out_type=out_type, mesh=_MESH,
                     scratch_types=tuple(scratch),
                     compiler_params=pltpu.CompilerParams(
                         use_tc_tiling_on_sc=False))


_agg = _make_agg()
_deg = _make_deg()

_BLK = 1000
_GRID = N // _BLK
_HIGH = lax.Precision.HIGHEST


def _mm_body(x_ref, w_ref, o_ref):
    o_ref[...] = jnp.dot(x_ref[...], w_ref[...], precision=_HIGH,
                         preferred_element_type=jnp.float32)


def _tc_matmul(x, w):
    return pl.pallas_call(
        _mm_body,
        grid=(_GRID,),
        in_specs=[pl.BlockSpec((_BLK, D), lambda i: (i, 0)),
                  pl.BlockSpec((D, D), lambda i: (0, 0))],
        out_specs=pl.BlockSpec((_BLK, D), lambda i: (i, 0)),
        out_shape=jax.ShapeDtypeStruct((N, D), jnp.float32),
    )(x, w)


def _combine_body(h_ref, p_ref, d0_ref, d1_ref, ws_ref, b_ref, wn_ref,
                  ho_ref, to_ref):
    deg = jnp.maximum(d0_ref[...] + d1_ref[...], 1.0)
    inv = 1.0 / deg[:, 0:1]
    hn = p_ref[...] * inv
    h = (jnp.dot(h_ref[...], ws_ref[...], precision=_HIGH,
                 preferred_element_type=jnp.float32) + hn + b_ref[...])
    ho_ref[...] = h
    to_ref[...] = jnp.dot(h, wn_ref[...], precision=_HIGH,
                          preferred_element_type=jnp.float32)


def _tc_combine(h, p, d0, d1, ws, b, wn):
    return pl.pallas_call(
        _combine_body,
        grid=(_GRID,),
        in_specs=[pl.BlockSpec((_BLK, D), lambda i: (i, 0)),
                  pl.BlockSpec((_BLK, D), lambda i: (i, 0)),
                  pl.BlockSpec((_BLK, DEG_W), lambda i: (i, 0)),
                  pl.BlockSpec((_BLK, DEG_W), lambda i: (i, 0)),
                  pl.BlockSpec((D, D), lambda i: (0, 0)),
                  pl.BlockSpec((1, D), lambda i: (0, 0)),
                  pl.BlockSpec((D, D), lambda i: (0, 0))],
        out_specs=[pl.BlockSpec((_BLK, D), lambda i: (i, 0)),
                   pl.BlockSpec((_BLK, D), lambda i: (i, 0))],
        out_shape=[jax.ShapeDtypeStruct((N, D), jnp.float32),
                   jax.ShapeDtypeStruct((N, D), jnp.float32)],
    )(h, p, d0, d1, ws, b.reshape(1, D), wn)


def _final_body(h_ref, p_ref, d0_ref, d1_ref, ws_ref, b_ref, ho_ref):
    deg = jnp.maximum(d0_ref[...] + d1_ref[...], 1.0)
    inv = 1.0 / deg[:, 0:1]
    hn = p_ref[...] * inv
    h = (jnp.dot(h_ref[...], ws_ref[...], precision=_HIGH,
                 preferred_element_type=jnp.float32) + hn + b_ref[...])
    ho_ref[...] = jnp.maximum(h, 0.0)


def _tc_final(h, p, d0, d1, ws, b):
    return pl.pallas_call(
        _final_body,
        grid=(_GRID,),
        in_specs=[pl.BlockSpec((_BLK, D), lambda i: (i, 0)),
                  pl.BlockSpec((_BLK, D), lambda i: (i, 0)),
                  pl.BlockSpec((_BLK, DEG_W), lambda i: (i, 0)),
                  pl.BlockSpec((_BLK, DEG_W), lambda i: (i, 0)),
                  pl.BlockSpec((D, D), lambda i: (0, 0)),
                  pl.BlockSpec((1, D), lambda i: (0, 0))],
        out_specs=pl.BlockSpec((_BLK, D), lambda i: (i, 0)),
        out_shape=jax.ShapeDtypeStruct((N, D), jnp.float32),
    )(h, p, d0, d1, ws, b.reshape(1, D))


def kernel(x, edge_index, W_self1, W_neigh1, b1, W_self2, W_neigh2, b2,
           W_self3, W_neigh3, b3):
    src = edge_index[0]
    dst = edge_index[1]
    pad = E_PAD - NUM_EDGES
    src_p = jnp.concatenate(
        [src, jnp.zeros((pad,), jnp.int32)]).reshape(NS, CHUNKS, CHUNK)
    dst_p = jnp.concatenate(
        [dst, jnp.full((pad,), N, jnp.int32)]).reshape(NS, CHUNKS, CHUNK)

    dg = _deg(dst_p)
    d0, d1 = dg[0, :N], dg[1, :N]

    def agg(t):
        p = _agg(t.reshape(2 * N, HD), src_p, dst_p)
        return p.reshape(N_PAD, D)[:N]

    t1 = _tc_matmul(x, W_neigh1)
    h1, t2 = _tc_combine(x, agg(t1), d0, d1, W_self1, b1, W_neigh2)
    h2, t3 = _tc_combine(h1, agg(t2), d0, d1, W_self2, b2, W_neigh3)
    return _tc_final(h2, agg(t3), d0, d1, W_self3, b3)
